# Initial kernel scaffold; baseline (speedup 1.0000x reference)
import jax
import jax.numpy as jnp
from jax.experimental import pallas as pl


def kernel(dr_vec, Z, neighbor_idxs, W):
    raise NotImplementedError("write your pallas kernel here")


# speedup vs baseline: 191.1684x; 191.1684x over previous
import jax
import jax.numpy as jnp
from jax import lax
from jax.experimental import pallas as pl
from jax.experimental.pallas import tpu as pltpu
from jax.experimental.pallas import tpu_sc as plsc

_E = 320000
_A = 10000
_AP = 10240
_DS = 128
_EB = 80
_NBLK = _E // _EB
_NTILES = 32
_BPT = _NBLK // _NTILES
_RPT = _AP // 16

_R_MAX = 6.0
_N_BASIS = 7
_BETTA = float(_N_BASIS * _N_BASIS) / (_R_MAX * _R_MAX)
_SHIFTS = [0.5 + i * (_R_MAX - 0.5) / (_N_BASIS - 1) for i in range(_N_BASIS)]
_PI = 3.14159265358979323846

_COS_COEF = [
    1.0, -0.5, 1.0 / 24, -1.0 / 720, 1.0 / 40320, -1.0 / 3628800,
    1.0 / 479001600, -1.0 / 87178291200, 1.0 / 20922789888000,
]

_PAIRS2 = [(i, j) for i in range(3) for j in range(i, 3)]
_TRIPLES3 = [(i, j, k) for i in range(3) for j in range(i, 3) for k in range(j, 3)]
_P2POS = {p: 4 + n for n, p in enumerate(_PAIRS2)}
_P3POS = {t: 10 + n for n, t in enumerate(_TRIPLES3)}


def _perms(t):
    import itertools
    return sorted(set(itertools.permutations(t)))


def _sc_edge_body(pk_h, ji_h, z_h, w_h, zeros_h, out_h,
                  pk0, pk1, ji0, ji1, jx0, jx1, rw0, rw1,
                  z_v, w_v, si0, si1, so0, so1, acc):
    c = lax.axis_index("c")
    s = lax.axis_index("s")
    wid = c * 16 + s

    pltpu.sync_copy(z_h, z_v)
    pltpu.sync_copy(w_h, w_v)
    pltpu.sync_copy(zeros_h.at[pl.ds(0, _EB)], rw0)
    pltpu.sync_copy(zeros_h.at[pl.ds(0, _EB)], rw1)
    pltpu.sync_copy(zeros_h, acc.at[pl.ds(s * _RPT, _RPT)])
    plsc.subcore_barrier()

    row0 = wid * _BPT
    iota = lax.iota(jnp.int32, 16)
    pk = [pk0, pk1]
    ji = [ji0, ji1]
    jx = [jx0, jx1]
    rw = [rw0, rw1]
    si = [si0, si1]
    so = [so0, so1]

    def prefetch(k, bb):
        pltpu.async_copy(pk_h.at[row0 + bb], pk[k], si[k])
        pltpu.async_copy(ji_h.at[row0 + bb], ji[k], si[k])

    def wait_in(k):
        pltpu.make_async_copy(pk_h.at[row0], pk[k], si[k]).wait()
        pltpu.make_async_copy(ji_h.at[row0], ji[k], si[k]).wait()

    def wait_out(k):
        pltpu.make_async_copy(rw[k], acc.at[jx[k]], so[k]).wait()

    def do_block(kb):
        pk_v, ji_v, jx_v, rows_v = pk[kb], ji[kb], jx[kb], rw[kb]
        for g in range(_EB // 16):
            sl = pl.ds(g * 16, 16)
            jx_v[sl] = ji_v[1, sl]
        for g in range(_EB // 16):
            sl = pl.ds(g * 16, 16)
            x = pk_v[0, sl]
            y = pk_v[1, sl]
            z = pk_v[2, sl]
            ii = ji_v[0, sl]
            jj = jx_v[sl]

            s2 = x * x + y * y + z * z
            yv = jnp.float32(2.0 ** 16)
            for k in range(-15, 9):
                yv = jnp.where(s2 >= jnp.float32(4.0 ** k),
                               jnp.float32(2.0 ** (-k - 1)), yv)
            for _ in range(5):
                yv = yv * (1.5 - 0.5 * s2 * yv * yv)
            dr = s2 * yv
            inv = 1.0 / (dr + 1e-5)
            dnx = x * inv
            dny = y * inv
            dnz = z * inv

            zi = plsc.load_gather(z_v, [ii])
            zj = plsc.load_gather(z_v, [jj])
            wbase = (zi * 10 + zj) * 35
            es = []
            for bi in range(_N_BASIS):
                t = dr - _SHIFTS[bi]
                es.append(jnp.exp(t * t * (-_BETTA)))
            radial = []
            for r in range(5):
                accv = es[0] * plsc.load_gather(w_v, [wbase + (r * 7)])
                for bi in range(1, _N_BASIS):
                    accv = accv + es[bi] * plsc.load_gather(
                        w_v, [wbase + (r * 7 + bi)])
                radial.append(accv)

            u = dr * (_PI / _R_MAX)
            u2 = u * u
            cosv = jnp.float32(_COS_COEF[-1])
            for ck in reversed(_COS_COEF[:-1]):
                cosv = cosv * u2 + ck
            cut = 0.5 * (cosv + 1.0)
            valid = (dr < _R_MAX) & (ii != jj)
            coef = jnp.where(valid, cut, 0.0)
            radial = [rv * coef for rv in radial]

            a1 = [dnx, dny, dnz]
            a2 = {}
            for (i, j) in _PAIRS2:
                a2[(i, j)] = a1[i] * a1[j]
            a3 = {}
            for (i, j, k) in _TRIPLES3:
                a3[(i, j, k)] = a2[(i, j)] * a1[k]

            erow = iota + (g * 16)
            zero16 = iota * 0
            for r in range(5):
                rv = radial[r]
                c0 = r * 20
                plsc.store_scatter(rows_v, [erow, zero16 + c0], rv)
                for i in range(3):
                    plsc.store_scatter(
                        rows_v, [erow, zero16 + (c0 + 1 + i)], rv * a1[i])
                for p in _PAIRS2:
                    plsc.store_scatter(
                        rows_v, [erow, zero16 + (c0 + _P2POS[p])], rv * a2[p])
                for t3 in _TRIPLES3:
                    plsc.store_scatter(
                        rows_v, [erow, zero16 + (c0 + _P3POS[t3])], rv * a3[t3])

        pltpu.async_copy(rows_v, acc.at[jx_v], so[kb], add=True)

    prefetch(0, 0)
    prefetch(1, 1)

    @pl.loop(0, (_BPT - 1) // 2)
    def _pair(p):
        for k in range(2):
            bb = p * 2 + k

            @pl.when(p > 0)
            def _wo():
                wait_out(k)

            wait_in(k)
            do_block(k)

            @pl.when(bb + 2 < _BPT)
            def _pf():
                prefetch(k, bb + 2)

    wait_out(0)
    wait_in(0)
    do_block(0)
    wait_out(1)
    wait_out(0)

    plsc.subcore_barrier()
    pltpu.sync_copy(acc.at[pl.ds(s * _RPT, _RPT)],
                    out_h.at[c].at[pl.ds(s * _RPT, _RPT)])


def _sc_moments(pk, jir, Z, Wf, zeros):
    mesh = plsc.VectorSubcoreMesh(core_axis_name="c", subcore_axis_name="s")
    return pl.kernel(
        _sc_edge_body,
        out_type=jax.ShapeDtypeStruct((2, _AP, _DS), jnp.float32),
        mesh=mesh,
        compiler_params=pltpu.CompilerParams(needs_layout_passes=False),
        scratch_types=[
            pltpu.VMEM((3, _EB), jnp.float32),
            pltpu.VMEM((3, _EB), jnp.float32),
            pltpu.VMEM((2, _EB), jnp.int32),
            pltpu.VMEM((2, _EB), jnp.int32),
            pltpu.VMEM((_EB,), jnp.int32),
            pltpu.VMEM((_EB,), jnp.int32),
            pltpu.VMEM((_EB, _DS), jnp.float32),
            pltpu.VMEM((_EB, _DS), jnp.float32),
            pltpu.VMEM((_A,), jnp.int32),
            pltpu.VMEM((3500,), jnp.float32),
            pltpu.SemaphoreType.DMA,
            pltpu.SemaphoreType.DMA,
            pltpu.SemaphoreType.DMA,
            pltpu.SemaphoreType.DMA,
            pltpu.VMEM_SHARED((_AP, _DS), jnp.float32),
        ],
    )(pk, jir, Z, Wf, zeros)


def _w2(i, j):
    return 1.0 if i == j else 2.0


def _w3(i, j, k):
    if i == j == k:
        return 1.0
    if i == j or j == k or i == k:
        return 3.0
    return 6.0


def _contract_body(p_ref, o_ref):
    m = p_ref[0] + p_ref[1]

    def col(f):
        return m[f]

    m0 = [col(r * 20) for r in range(5)]
    m1 = [[col(r * 20 + 1 + i) for i in range(3)] for r in range(5)]

    def c2(r, i, j):
        return col(r * 20 + _P2POS[tuple(sorted((i, j)))])

    def c3(r, i, j, k):
        return col(r * 20 + _P3POS[tuple(sorted((i, j, k)))])

    m2 = [[[c2(r, i, j) for j in range(3)] for i in range(3)]
          for r in range(5)]
    m3 = [[[[c3(r, i, j, k) for k in range(3)]
            for j in range(3)] for i in range(3)] for r in range(5)]

    outs = [None] * 580

    for r in range(5):
        outs[r] = m0[r]

    m2w = {(r, p): m2[r][p[0]][p[1]] * _w2(*p) if _w2(*p) != 1.0
           else m2[r][p[0]][p[1]]
           for r in range(5) for p in _PAIRS2}
    m3w = {(r, t): m3[r][t[0]][t[1]][t[2]] * _w3(*t) if _w3(*t) != 1.0
           else m3[r][t[0]][t[1]][t[2]]
           for r in range(5) for t in _TRIPLES3}

    for r in range(5):
        for s in range(r, 5):
            v = m1[r][0] * m1[s][0]
            for i in range(1, 3):
                v = v + m1[r][i] * m1[s][i]
            outs[5 + r * 5 + s] = v
            outs[5 + s * 5 + r] = v

    for r in range(5):
        for s in range(r, 5):
            v = None
            for p in _PAIRS2:
                term = m2w[(r, p)] * m2[s][p[0]][p[1]]
                v = term if v is None else v + term
            outs[30 + r * 5 + s] = v
            outs[30 + s * 5 + r] = v

    for r in range(5):
        for s in range(r, 5):
            v = None
            for t in _TRIPLES3:
                term = m3w[(r, t)] * m3[s][t[0]][t[1]][t[2]]
                v = term if v is None else v + term
            outs[55 + r * 5 + s] = v
            outs[55 + s * 5 + r] = v

    A = {}
    for r in range(5):
        for t in range(5):
            for j in range(3):
                v = m2[r][0][j] * m1[t][0]
                for i in range(1, 3):
                    v = v + m2[r][i][j] * m1[t][i]
                A[(r, t, j)] = v
    for r in range(5):
        for s in range(5):
            for t in range(s, 5):
                v = A[(r, t, 0)] * m1[s][0]
                for j in range(1, 3):
                    v = v + A[(r, t, j)] * m1[s][j]
                outs[80 + r * 25 + s * 5 + t] = v
                outs[80 + r * 25 + t * 5 + s] = v

    Dd = {}
    for r in range(5):
        for s in range(5):
            for k in range(3):
                v = None
                for p in _PAIRS2:
                    term = m3[r][p[0]][p[1]][k] * m2w[(s, p)]
                    v = term if v is None else v + term
                Dd[(r, s, k)] = v
    for r in range(5):
        for s in range(5):
            for t in range(5):
                v = Dd[(r, s, 0)] * m1[t][0]
                for k in range(1, 3):
                    v = v + Dd[(r, s, k)] * m1[t][k]
                outs[205 + r * 25 + s * 5 + t] = v

    Cc = {}
    for r in range(5):
        for s in range(r, 5):
            for j in range(3):
                for k in range(3):
                    v = m2[r][0][j] * m2[s][0][k]
                    for i in range(1, 3):
                        v = v + m2[r][i][j] * m2[s][i][k]
                    Cc[(r, s, j, k)] = v
    for r in range(5):
        for s in range(r, 5):
            for t in range(s, 5):
                v = None
                for j in range(3):
                    for k in range(3):
                        term = Cc[(r, s, j, k)] * m2[t][j][k]
                        v = term if v is None else v + term
                for (p, q, u) in _perms((r, s, t)):
                    outs[330 + p * 25 + q * 5 + u] = v

    Ee = {}
    for r in range(5):
        for s in range(r, 5):
            for k in range(3):
                for li in range(3):
                    v = None
                    for (i, j) in _PAIRS2:
                        term = m3[r][i][j][k] * m3[s][i][j][li]
                        if i != j:
                            term = term + term
                        v = term if v is None else v + term
                    Ee[(r, s, k, li)] = v
    for r in range(5):
        for s in range(r, 5):
            for t in range(5):
                v = None
                for k in range(3):
                    for li in range(3):
                        term = Ee[(r, s, k, li)] * m2[t][k][li]
                        v = term if v is None else v + term
                outs[455 + r * 25 + s * 5 + t] = v
                outs[455 + s * 25 + r * 5 + t] = v

    for f in range(580):
        o_ref[f] = outs[f]


def _tc_contract(pt, interpret=False):
    nb = _AP // 1024
    return pl.pallas_call(
        _contract_body,
        out_shape=jax.ShapeDtypeStruct((580, nb * 8, 128), jnp.float32),
        grid=(nb,),
        in_specs=[pl.BlockSpec((2, _DS, 8, 128), lambda i: (0, 0, i, 0))],
        out_specs=pl.BlockSpec((580, 8, 128), lambda i: (0, i, 0)),
        compiler_params=pltpu.CompilerParams(
            dimension_semantics=("parallel",)),
        interpret=interpret,
    )(pt)


def kernel(dr_vec, Z, neighbor_idxs, W):
    pk = dr_vec.T.reshape(3, _NBLK, _EB).transpose(1, 0, 2)
    jir = neighbor_idxs.reshape(2, _NBLK, _EB).transpose(1, 0, 2)
    Wf = W.reshape(-1)
    zeros = jnp.zeros((_RPT, _DS), jnp.float32)

    partials = _sc_moments(pk, jir, Z, Wf, zeros)
    pt = partials.transpose(0, 2, 1).reshape(2, _DS, _AP // 128, 128)
    out_t = _tc_contract(pt)
    out = out_t.reshape(580, _AP).T
    return out[:_A]


# speedup vs baseline: 193.0725x; 1.0100x over previous
import jax
import jax.numpy as jnp
from jax import lax
from jax.experimental import pallas as pl
from jax.experimental.pallas import tpu as pltpu
from jax.experimental.pallas import tpu_sc as plsc

_E = 320000
_A = 10000
_AP = 10240
_DS = 128
_EB = 80
_NBLK = _E // _EB
_NTILES = 32
_BPT = _NBLK // _NTILES
_RPT = _AP // 16

_R_MAX = 6.0
_N_BASIS = 7
_BETTA = float(_N_BASIS * _N_BASIS) / (_R_MAX * _R_MAX)
_SHIFTS = [0.5 + i * (_R_MAX - 0.5) / (_N_BASIS - 1) for i in range(_N_BASIS)]
_PI = 3.14159265358979323846

_COS_COEF = [
    1.0, -0.5, 1.0 / 24, -1.0 / 720, 1.0 / 40320, -1.0 / 3628800,
    1.0 / 479001600, -1.0 / 87178291200,
]

_PAIRS2 = [(i, j) for i in range(3) for j in range(i, 3)]
_TRIPLES3 = [(i, j, k) for i in range(3) for j in range(i, 3) for k in range(j, 3)]
_P2POS = {p: 4 + n for n, p in enumerate(_PAIRS2)}
_P3POS = {t: 10 + n for n, t in enumerate(_TRIPLES3)}


def _perms(t):
    import itertools
    return sorted(set(itertools.permutations(t)))


def _sc_edge_body(pk_h, ji_h, z_h, w_h, zeros_h, out_h,
                  pk0, pk1, ji0, ji1, jx0, jx1, rw0, rw1,
                  z_v, w_v, si0, si1, so0, so1, acc):
    c = lax.axis_index("c")
    s = lax.axis_index("s")
    wid = c * 16 + s

    pltpu.sync_copy(z_h, z_v)
    pltpu.sync_copy(w_h, w_v)
    pltpu.sync_copy(zeros_h.at[pl.ds(0, _EB)], rw0)
    pltpu.sync_copy(zeros_h.at[pl.ds(0, _EB)], rw1)
    pltpu.sync_copy(zeros_h, acc.at[pl.ds(s * _RPT, _RPT)])
    plsc.subcore_barrier()

    row0 = wid * _BPT
    iota = lax.iota(jnp.int32, 16)
    pk = [pk0, pk1]
    ji = [ji0, ji1]
    jx = [jx0, jx1]
    rw = [rw0, rw1]
    si = [si0, si1]
    so = [so0, so1]

    def prefetch(k, bb):
        pltpu.async_copy(pk_h.at[row0 + bb], pk[k], si[k])
        pltpu.async_copy(ji_h.at[row0 + bb], ji[k], si[k])

    def wait_in(k):
        pltpu.make_async_copy(pk_h.at[row0], pk[k], si[k]).wait()
        pltpu.make_async_copy(ji_h.at[row0], ji[k], si[k]).wait()

    def wait_out(k):
        pltpu.make_async_copy(rw[k], acc.at[jx[k]], so[k]).wait()

    def do_block(kb):
        pk_v, ji_v, jx_v, rows_v = pk[kb], ji[kb], jx[kb], rw[kb]
        for g in range(_EB // 16):
            sl = pl.ds(g * 16, 16)
            jx_v[sl] = ji_v[1, sl]
        for g in range(_EB // 16):
            sl = pl.ds(g * 16, 16)
            x = pk_v[0, sl]
            y = pk_v[1, sl]
            z = pk_v[2, sl]
            ii = ji_v[0, sl]
            jj = jx_v[sl]

            s2 = x * x + y * y + z * z
            yv = jnp.float32(2.0 ** 16)
            for k in range(-15, 9):
                yv = jnp.where(s2 >= jnp.float32(4.0 ** k),
                               jnp.float32(2.0 ** (-k - 1)), yv)
            for _ in range(5):
                yv = yv * (1.5 - 0.5 * s2 * yv * yv)
            dr = s2 * yv
            inv = 1.0 / (dr + 1e-5)
            dnx = x * inv
            dny = y * inv
            dnz = z * inv

            zi = plsc.load_gather(z_v, [ii])
            zj = plsc.load_gather(z_v, [jj])
            wbase = (zi * 10 + zj) * 35
            es = []
            for bi in range(_N_BASIS):
                t = dr - _SHIFTS[bi]
                es.append(jnp.exp(t * t * (-_BETTA)))
            radial = []
            for r in range(5):
                accv = es[0] * plsc.load_gather(w_v, [wbase + (r * 7)])
                for bi in range(1, _N_BASIS):
                    accv = accv + es[bi] * plsc.load_gather(
                        w_v, [wbase + (r * 7 + bi)])
                radial.append(accv)

            u = dr * (_PI / _R_MAX)
            u2 = u * u
            cosv = jnp.float32(_COS_COEF[-1])
            for ck in reversed(_COS_COEF[:-1]):
                cosv = cosv * u2 + ck
            cut = 0.5 * (cosv + 1.0)
            valid = (dr < _R_MAX) & (ii != jj)
            coef = jnp.where(valid, cut, 0.0)
            radial = [rv * coef for rv in radial]

            a1 = [dnx, dny, dnz]
            erow = iota + (g * 16)
            zero16 = iota * 0
            for r in range(5):
                rv = radial[r]
                c0 = r * 20
                plsc.store_scatter(rows_v, [erow, zero16 + c0], rv)
                v1 = []
                for i in range(3):
                    v1.append(rv * a1[i])
                    plsc.store_scatter(
                        rows_v, [erow, zero16 + (c0 + 1 + i)], v1[i])
                v2 = {}
                for p in _PAIRS2:
                    v2[p] = v1[p[0]] * a1[p[1]]
                    plsc.store_scatter(
                        rows_v, [erow, zero16 + (c0 + _P2POS[p])], v2[p])
                for t3 in _TRIPLES3:
                    plsc.store_scatter(
                        rows_v, [erow, zero16 + (c0 + _P3POS[t3])],
                        v2[t3[:2]] * a1[t3[2]])

        pltpu.async_copy(rows_v, acc.at[jx_v], so[kb], add=True)

    prefetch(0, 0)
    prefetch(1, 1)

    @pl.loop(0, (_BPT - 1) // 2)
    def _pair(p):
        for k in range(2):
            bb = p * 2 + k

            @pl.when(p > 0)
            def _wo():
                wait_out(k)

            wait_in(k)
            do_block(k)

            @pl.when(bb + 2 < _BPT)
            def _pf():
                prefetch(k, bb + 2)

    wait_out(0)
    wait_in(0)
    do_block(0)
    wait_out(1)
    wait_out(0)

    plsc.subcore_barrier()
    pltpu.sync_copy(acc.at[pl.ds(s * _RPT, _RPT)],
                    out_h.at[c].at[pl.ds(s * _RPT, _RPT)])


def _sc_moments(pk, jir, Z, Wf, zeros):
    mesh = plsc.VectorSubcoreMesh(core_axis_name="c", subcore_axis_name="s")
    return pl.kernel(
        _sc_edge_body,
        out_type=jax.ShapeDtypeStruct((2, _AP, _DS), jnp.float32),
        mesh=mesh,
        compiler_params=pltpu.CompilerParams(needs_layout_passes=False),
        scratch_types=[
            pltpu.VMEM((3, _EB), jnp.float32),
            pltpu.VMEM((3, _EB), jnp.float32),
            pltpu.VMEM((2, _EB), jnp.int32),
            pltpu.VMEM((2, _EB), jnp.int32),
            pltpu.VMEM((_EB,), jnp.int32),
            pltpu.VMEM((_EB,), jnp.int32),
            pltpu.VMEM((_EB, _DS), jnp.float32),
            pltpu.VMEM((_EB, _DS), jnp.float32),
            pltpu.VMEM((_A,), jnp.int32),
            pltpu.VMEM((3500,), jnp.float32),
            pltpu.SemaphoreType.DMA,
            pltpu.SemaphoreType.DMA,
            pltpu.SemaphoreType.DMA,
            pltpu.SemaphoreType.DMA,
            pltpu.VMEM_SHARED((_AP, _DS), jnp.float32),
        ],
    )(pk, jir, Z, Wf, zeros)


def _w2(i, j):
    return 1.0 if i == j else 2.0


def _w3(i, j, k):
    if i == j == k:
        return 1.0
    if i == j or j == k or i == k:
        return 3.0
    return 6.0


def _contract_body(p_ref, o_ref):
    m = p_ref[0] + p_ref[1]

    def col(f):
        return m[f]

    m0 = [col(r * 20) for r in range(5)]
    m1 = [[col(r * 20 + 1 + i) for i in range(3)] for r in range(5)]

    def c2(r, i, j):
        return col(r * 20 + _P2POS[tuple(sorted((i, j)))])

    def c3(r, i, j, k):
        return col(r * 20 + _P3POS[tuple(sorted((i, j, k)))])

    m2 = [[[c2(r, i, j) for j in range(3)] for i in range(3)]
          for r in range(5)]
    m3 = [[[[c3(r, i, j, k) for k in range(3)]
            for j in range(3)] for i in range(3)] for r in range(5)]

    outs = [None] * 580

    for r in range(5):
        outs[r] = m0[r]

    m2w = {(r, p): m2[r][p[0]][p[1]] * _w2(*p) if _w2(*p) != 1.0
           else m2[r][p[0]][p[1]]
           for r in range(5) for p in _PAIRS2}
    m3w = {(r, t): m3[r][t[0]][t[1]][t[2]] * _w3(*t) if _w3(*t) != 1.0
           else m3[r][t[0]][t[1]][t[2]]
           for r in range(5) for t in _TRIPLES3}

    for r in range(5):
        for s in range(r, 5):
            v = m1[r][0] * m1[s][0]
            for i in range(1, 3):
                v = v + m1[r][i] * m1[s][i]
            outs[5 + r * 5 + s] = v
            outs[5 + s * 5 + r] = v

    for r in range(5):
        for s in range(r, 5):
            v = None
            for p in _PAIRS2:
                term = m2w[(r, p)] * m2[s][p[0]][p[1]]
                v = term if v is None else v + term
            outs[30 + r * 5 + s] = v
            outs[30 + s * 5 + r] = v

    for r in range(5):
        for s in range(r, 5):
            v = None
            for t in _TRIPLES3:
                term = m3w[(r, t)] * m3[s][t[0]][t[1]][t[2]]
                v = term if v is None else v + term
            outs[55 + r * 5 + s] = v
            outs[55 + s * 5 + r] = v

    A = {}
    for r in range(5):
        for t in range(5):
            for j in range(3):
                v = m2[r][0][j] * m1[t][0]
                for i in range(1, 3):
                    v = v + m2[r][i][j] * m1[t][i]
                A[(r, t, j)] = v
    for r in range(5):
        for s in range(5):
            for t in range(s, 5):
                v = A[(r, t, 0)] * m1[s][0]
                for j in range(1, 3):
                    v = v + A[(r, t, j)] * m1[s][j]
                outs[80 + r * 25 + s * 5 + t] = v
                outs[80 + r * 25 + t * 5 + s] = v

    Dd = {}
    for r in range(5):
        for s in range(5):
            for k in range(3):
                v = None
                for p in _PAIRS2:
                    term = m3[r][p[0]][p[1]][k] * m2w[(s, p)]
                    v = term if v is None else v + term
                Dd[(r, s, k)] = v
    for r in range(5):
        for s in range(5):
            for t in range(5):
                v = Dd[(r, s, 0)] * m1[t][0]
                for k in range(1, 3):
                    v = v + Dd[(r, s, k)] * m1[t][k]
                outs[205 + r * 25 + s * 5 + t] = v

    Cc = {}
    for r in range(5):
        for s in range(r, 5):
            for j in range(3):
                for k in range(3):
                    v = m2[r][0][j] * m2[s][0][k]
                    for i in range(1, 3):
                        v = v + m2[r][i][j] * m2[s][i][k]
                    Cc[(r, s, j, k)] = v
    for r in range(5):
        for s in range(r, 5):
            for t in range(s, 5):
                v = None
                for j in range(3):
                    for k in range(3):
                        term = Cc[(r, s, j, k)] * m2[t][j][k]
                        v = term if v is None else v + term
                for (p, q, u) in _perms((r, s, t)):
                    outs[330 + p * 25 + q * 5 + u] = v

    Ee = {}
    for r in range(5):
        for s in range(r, 5):
            for k in range(3):
                for li in range(3):
                    v = None
                    for (i, j) in _PAIRS2:
                        term = m3[r][i][j][k] * m3[s][i][j][li]
                        if i != j:
                            term = term + term
                        v = term if v is None else v + term
                    Ee[(r, s, k, li)] = v
    for r in range(5):
        for s in range(r, 5):
            for t in range(5):
                v = None
                for k in range(3):
                    for li in range(3):
                        term = Ee[(r, s, k, li)] * m2[t][k][li]
                        v = term if v is None else v + term
                outs[455 + r * 25 + s * 5 + t] = v
                outs[455 + s * 25 + r * 5 + t] = v

    for f in range(580):
        o_ref[f] = outs[f]


def _tc_contract(pt, interpret=False):
    nb = _AP // 1024
    return pl.pallas_call(
        _contract_body,
        out_shape=jax.ShapeDtypeStruct((580, nb * 8, 128), jnp.float32),
        grid=(nb,),
        in_specs=[pl.BlockSpec((2, _DS, 8, 128), lambda i: (0, 0, i, 0))],
        out_specs=pl.BlockSpec((580, 8, 128), lambda i: (0, i, 0)),
        compiler_params=pltpu.CompilerParams(
            dimension_semantics=("parallel",)),
        interpret=interpret,
    )(pt)


def kernel(dr_vec, Z, neighbor_idxs, W):
    pk = dr_vec.T.reshape(3, _NBLK, _EB).transpose(1, 0, 2)
    jir = neighbor_idxs.reshape(2, _NBLK, _EB).transpose(1, 0, 2)
    Wf = W.reshape(-1)
    zeros = jnp.zeros((_RPT, _DS), jnp.float32)

    partials = _sc_moments(pk, jir, Z, Wf, zeros)
    pt = partials.transpose(0, 2, 1).reshape(2, _DS, _AP // 128, 128)
    out_t = _tc_contract(pt)
    out = out_t.reshape(580, _AP).T
    return out[:_A]
